# baseline (speedup 1.0000x reference)
import dataclasses
import functools

import jax
import jax.numpy as jnp
from jax import lax
from jax.experimental import pallas as pl
from jax.experimental.pallas import tpu as pltpu
from jax.experimental.pallas import tpu_sc as plsc

_BATCH = 16384
_GOAL_DIM = 16
_NODES = 1_000_000
_NUM_WORKERS = 32
_B_PER_W = _BATCH // _NUM_WORKERS
_CHUNK = 128
_LANES = 16


def kernel(node_ids, weight):
    idx = node_ids.astype(jnp.int32)
    grouped = weight.reshape(_NODES // 8, 128)
    mesh = plsc.VectorSubcoreMesh(core_axis_name="c", subcore_axis_name="s")
    cp = pltpu.CompilerParams()
    if "needs_layout_passes" in pltpu.CompilerParams.__dataclass_fields__:
        cp = dataclasses.replace(cp, needs_layout_passes=False)

    @functools.partial(
        pl.kernel,
        mesh=mesh,
        compiler_params=cp,
        out_type=jax.ShapeDtypeStruct((_GOAL_DIM, _BATCH), jnp.float32),
        scratch_types=[
            pltpu.VMEM((_B_PER_W,), jnp.int32),
            pltpu.VMEM((_B_PER_W,), jnp.int32),
            pltpu.VMEM((_B_PER_W, 128), jnp.float32),
            pltpu.VMEM((_GOAL_DIM, _B_PER_W), jnp.float32),
            pltpu.SemaphoreType.DMA,
        ],
    )
    def _gather(g_hbm, i_hbm, o_hbm, idx_v, grp_v, blk_v, out_v, sem):
        wid = lax.axis_index("c") * 16 + lax.axis_index("s")
        base = wid * _B_PER_W

        pltpu.sync_copy(i_hbm.at[pl.ds(base, _B_PER_W)], idx_v)

        @pl.loop(0, _B_PER_W, step=_LANES)
        def _(c):
            grp_v[pl.ds(c, _LANES)] = idx_v[pl.ds(c, _LANES)] >> 3

        @pl.loop(0, _B_PER_W, step=_CHUNK)
        def _(c):
            pltpu.async_copy(
                g_hbm.at[grp_v.at[pl.ds(c, _CHUNK)]],
                blk_v.at[pl.ds(c, _CHUNK), :],
                sem,
            )

        @pl.loop(0, _B_PER_W, step=_CHUNK)
        def _(c):
            pltpu.make_async_copy(
                g_hbm.at[grp_v.at[pl.ds(c, _CHUNK)]],
                blk_v.at[pl.ds(c, _CHUNK), :],
                sem,
            ).wait()

        lane = lax.iota(jnp.int32, _LANES)

        @pl.loop(0, _B_PER_W, step=_LANES)
        def _(c):
            b_vec = c + lane
            sub16 = (idx_v[pl.ds(c, _LANES)] & 7) * 16
            for l in range(_LANES):
                out_v[l, pl.ds(c, _LANES)] = plsc.load_gather(
                    blk_v, [b_vec, sub16 + l]
                )

        pltpu.sync_copy(out_v, o_hbm.at[:, pl.ds(base, _B_PER_W)])

    out_t = _gather(grouped, idx)
    return out_t.T


# speedup vs baseline: 1.0324x; 1.0324x over previous
import dataclasses
import functools

import jax
import jax.numpy as jnp
from jax import lax
from jax.experimental import pallas as pl
from jax.experimental.pallas import tpu as pltpu
from jax.experimental.pallas import tpu_sc as plsc

_BATCH = 16384
_GOAL_DIM = 16
_NODES = 1_000_000
_NUM_WORKERS = 32
_B_PER_W = _BATCH // _NUM_WORKERS
_CHUNK = 128
_LANES = 16
_GROUPS = _NODES // 8
_SLABS = (_NODES + 127) // 128
_SLABS_PER_W = (_SLABS + _NUM_WORKERS - 1) // _NUM_WORKERS


def _mesh_and_params():
    mesh = plsc.VectorSubcoreMesh(core_axis_name="c", subcore_axis_name="s")
    cp = pltpu.CompilerParams()
    if "needs_layout_passes" in pltpu.CompilerParams.__dataclass_fields__:
        cp = dataclasses.replace(cp, needs_layout_passes=False)
    return mesh, cp


def kernel(node_ids, weight):
    idx = node_ids.astype(jnp.int32)
    w_t = weight.T
    mesh, cp = _mesh_and_params()

    @functools.partial(
        pl.kernel,
        mesh=mesh,
        compiler_params=cp,
        out_type=jax.ShapeDtypeStruct((_GROUPS, 128), jnp.float32),
        scratch_types=[
            pltpu.VMEM((_GOAL_DIM, 128), jnp.float32),
            pltpu.VMEM((_GOAL_DIM, 128), jnp.float32),
            pltpu.VMEM((_GOAL_DIM, 128), jnp.float32),
            pltpu.SemaphoreType.DMA,
        ],
    )
    def _relayout(w_hbm, g_hbm, in0, in1, out_b, sem_in):
        wid = lax.axis_index("c") * 16 + lax.axis_index("s")
        k0 = wid * _SLABS_PER_W
        n_full = jnp.minimum(
            jnp.maximum(_SLABS - 1 - k0, 0), _SLABS_PER_W
        )
        lane = lax.iota(jnp.int32, _LANES)
        bufs = (in0, in1)

        def start(k, buf):
            pltpu.async_copy(
                w_hbm.at[:, pl.ds((k0 + k) * 128, 128)], buf, sem_in
            )

        def wait(k, buf):
            pltpu.make_async_copy(
                w_hbm.at[:, pl.ds((k0 + k) * 128, 128)], buf, sem_in
            ).wait()

        def transpose_slab(buf):
            for g in range(16):
                for s in range(8):
                    out_b[g, pl.ds(s * _LANES, _LANES)] = plsc.load_gather(
                        buf, [lane, jnp.full((_LANES,), 8 * g + s, jnp.int32)]
                    )

        @pl.when(n_full > 0)
        def _():
            start(0, bufs[0])

            @pl.loop(0, _SLABS_PER_W)
            def _(k):
                @pl.when(k < n_full)
                def _():
                    @pl.when(lax.rem(k, 2) == 0)
                    def _():
                        wait(k, bufs[0])

                        @pl.when(k + 1 < n_full)
                        def _():
                            start(k + 1, bufs[1])

                        transpose_slab(bufs[0])

                    @pl.when(lax.rem(k, 2) == 1)
                    def _():
                        wait(k, bufs[1])

                        @pl.when(k + 1 < n_full)
                        def _():
                            start(k + 1, bufs[0])

                        transpose_slab(bufs[1])

                    pltpu.sync_copy(
                        out_b, g_hbm.at[pl.ds((k0 + k) * 16, 16), :]
                    )


    @functools.partial(
        pl.kernel,
        mesh=mesh,
        compiler_params=cp,
        out_type=jax.ShapeDtypeStruct((_GOAL_DIM, _BATCH), jnp.float32),
        scratch_types=[
            pltpu.VMEM((_B_PER_W,), jnp.int32),
            pltpu.VMEM((_B_PER_W,), jnp.int32),
            pltpu.VMEM((_B_PER_W, 128), jnp.float32),
            pltpu.VMEM((_GOAL_DIM, _B_PER_W), jnp.float32),
            pltpu.VMEM((8, 128), jnp.float32),
            pltpu.SemaphoreType.DMA,
        ],
    )
    def _gather(g_hbm, t_hbm, i_hbm, o_hbm, idx_v, grp_v, blk_v, out_v,
                tail_sp, sem):
        wid = lax.axis_index("c") * 16 + lax.axis_index("s")
        base = wid * _B_PER_W

        pltpu.sync_copy(t_hbm, tail_sp)
        pltpu.sync_copy(i_hbm.at[pl.ds(base, _B_PER_W)], idx_v)

        @pl.loop(0, _B_PER_W, step=_LANES)
        def _(c):
            grp_v[pl.ds(c, _LANES)] = idx_v[pl.ds(c, _LANES)] >> 3

        @pl.loop(0, _B_PER_W, step=_CHUNK)
        def _(c):
            pltpu.async_copy(
                g_hbm.at[grp_v.at[pl.ds(c, _CHUNK)]],
                blk_v.at[pl.ds(c, _CHUNK), :],
                sem,
            )

        @pl.loop(0, _B_PER_W, step=_CHUNK)
        def _(c):
            pltpu.make_async_copy(
                g_hbm.at[grp_v.at[pl.ds(c, _CHUNK)]],
                blk_v.at[pl.ds(c, _CHUNK), :],
                sem,
            ).wait()

        lane = lax.iota(jnp.int32, _LANES)

        @pl.loop(0, _B_PER_W, step=_LANES)
        def _(c):
            b_vec = c + lane
            grp_c = grp_v[pl.ds(c, _LANES)]
            is_tail = grp_c >= _GROUPS - 8
            tg = jnp.where(is_tail, grp_c - (_GROUPS - 8), 0)
            sub16 = (idx_v[pl.ds(c, _LANES)] & 7) * 16
            for l in range(_LANES):
                main = plsc.load_gather(blk_v, [b_vec, sub16 + l])
                tailv = plsc.load_gather(tail_sp, [tg, sub16 + l])
                out_v[l, pl.ds(c, _LANES)] = jnp.where(is_tail, tailv, main)

        pltpu.sync_copy(out_v, o_hbm.at[:, pl.ds(base, _B_PER_W)])

    tail_tbl = weight[_NODES - 64:].reshape(8, 128)
    grouped = _relayout(w_t)
    out_t = _gather(grouped, tail_tbl, idx)
    return out_t.T


# speedup vs baseline: 1.1174x; 1.0824x over previous
import dataclasses
import functools

import jax
import jax.numpy as jnp
from jax import lax
from jax.experimental import pallas as pl
from jax.experimental.pallas import tpu as pltpu
from jax.experimental.pallas import tpu_sc as plsc

_BATCH = 16384
_GOAL_DIM = 16
_NODES = 1_000_000
_NUM_WORKERS = 32
_B_PER_W = _BATCH // _NUM_WORKERS
_CHUNK = 128
_LANES = 16
_GROUPS = _NODES // 8
_SLABS = (_NODES + 127) // 128
_SLABS_PER_W = (_SLABS + _NUM_WORKERS - 1) // _NUM_WORKERS


def _mesh_and_params():
    mesh = plsc.VectorSubcoreMesh(core_axis_name="c", subcore_axis_name="s")
    cp = pltpu.CompilerParams()
    if "needs_layout_passes" in pltpu.CompilerParams.__dataclass_fields__:
        cp = dataclasses.replace(cp, needs_layout_passes=False)
    return mesh, cp


def kernel(node_ids, weight):
    idx = node_ids.astype(jnp.int32)
    w_t = weight.T
    mesh, cp = _mesh_and_params()

    @functools.partial(
        pl.kernel,
        mesh=mesh,
        compiler_params=cp,
        out_type=jax.ShapeDtypeStruct((_GROUPS, 128), jnp.float32),
        scratch_types=[
            pltpu.VMEM((_GOAL_DIM, 128), jnp.float32),
            pltpu.VMEM((_GOAL_DIM, 128), jnp.float32),
            pltpu.VMEM((_GOAL_DIM, 128), jnp.float32),
            pltpu.VMEM((_GOAL_DIM, 128), jnp.float32),
            pltpu.SemaphoreType.DMA,
            pltpu.SemaphoreType.DMA,
        ],
    )
    def _relayout(w_hbm, g_hbm, in0, in1, out0, out1, sem_in, sem_out):
        wid = lax.axis_index("c") * 16 + lax.axis_index("s")
        k0 = wid * _SLABS_PER_W
        n_full = jnp.minimum(
            jnp.maximum(_SLABS - 1 - k0, 0), _SLABS_PER_W
        )
        lane = lax.iota(jnp.int32, _LANES)
        ins = (in0, in1)
        outs = (out0, out1)

        def start_in(k, buf):
            pltpu.async_copy(
                w_hbm.at[:, pl.ds((k0 + k) * 128, 128)], buf, sem_in
            )

        def wait_in(k, buf):
            pltpu.make_async_copy(
                w_hbm.at[:, pl.ds((k0 + k) * 128, 128)], buf, sem_in
            ).wait()

        def start_out(k, buf):
            pltpu.async_copy(
                buf, g_hbm.at[pl.ds((k0 + k) * 16, 16), :], sem_out
            )

        def wait_out(k, buf):
            pltpu.make_async_copy(
                buf, g_hbm.at[pl.ds((k0 + k) * 16, 16), :], sem_out
            ).wait()

        def transpose_slab(buf, out_b):
            for g in range(16):
                for s in range(8):
                    out_b[g, pl.ds(s * _LANES, _LANES)] = plsc.load_gather(
                        buf, [lane, jnp.full((_LANES,), 8 * g + s, jnp.int32)]
                    )

        def step(k, p):
            @pl.when(k < n_full)
            def _():
                wait_in(k, ins[p])

                @pl.when(k + 2 < n_full)
                def _():
                    start_in(k + 2, ins[p])

                @pl.when(k >= 2)
                def _():
                    wait_out(k - 2, outs[p])

                transpose_slab(ins[p], outs[p])
                start_out(k, outs[p])

        @pl.when(n_full > 0)
        def _():
            start_in(0, ins[0])

            @pl.when(n_full > 1)
            def _():
                start_in(1, ins[1])

            @pl.loop(0, _SLABS_PER_W, step=2)
            def _(k):
                step(k, 0)
                step(k + 1, 1)

            @pl.when(n_full >= 2)
            def _():
                wait_out(n_full - 2, outs[1])

            @pl.when(n_full >= 1)
            def _():
                wait_out(n_full - 1, outs[0])


    @functools.partial(
        pl.kernel,
        mesh=mesh,
        compiler_params=cp,
        out_type=jax.ShapeDtypeStruct((_GOAL_DIM, _BATCH), jnp.float32),
        scratch_types=[
            pltpu.VMEM((_B_PER_W,), jnp.int32),
            pltpu.VMEM((_B_PER_W,), jnp.int32),
            pltpu.VMEM((_B_PER_W, 128), jnp.float32),
            pltpu.VMEM((_GOAL_DIM, _B_PER_W), jnp.float32),
            pltpu.VMEM((8, 128), jnp.float32),
            pltpu.SemaphoreType.DMA,
        ],
    )
    def _gather(g_hbm, t_hbm, i_hbm, o_hbm, idx_v, grp_v, blk_v, out_v,
                tail_sp, sem):
        wid = lax.axis_index("c") * 16 + lax.axis_index("s")
        base = wid * _B_PER_W

        pltpu.sync_copy(t_hbm, tail_sp)
        pltpu.sync_copy(i_hbm.at[pl.ds(base, _B_PER_W)], idx_v)

        @pl.loop(0, _B_PER_W, step=_LANES)
        def _(c):
            grp_v[pl.ds(c, _LANES)] = idx_v[pl.ds(c, _LANES)] >> 3

        @pl.loop(0, _B_PER_W, step=_CHUNK)
        def _(c):
            pltpu.async_copy(
                g_hbm.at[grp_v.at[pl.ds(c, _CHUNK)]],
                blk_v.at[pl.ds(c, _CHUNK), :],
                sem,
            )

        @pl.loop(0, _B_PER_W, step=_CHUNK)
        def _(c):
            pltpu.make_async_copy(
                g_hbm.at[grp_v.at[pl.ds(c, _CHUNK)]],
                blk_v.at[pl.ds(c, _CHUNK), :],
                sem,
            ).wait()

        lane = lax.iota(jnp.int32, _LANES)

        @pl.loop(0, _B_PER_W, step=_LANES)
        def _(c):
            b_vec = c + lane
            grp_c = grp_v[pl.ds(c, _LANES)]
            is_tail = grp_c >= _GROUPS - 8
            tg = jnp.where(is_tail, grp_c - (_GROUPS - 8), 0)
            sub16 = (idx_v[pl.ds(c, _LANES)] & 7) * 16
            for l in range(_LANES):
                main = plsc.load_gather(blk_v, [b_vec, sub16 + l])
                tailv = plsc.load_gather(tail_sp, [tg, sub16 + l])
                out_v[l, pl.ds(c, _LANES)] = jnp.where(is_tail, tailv, main)

        pltpu.sync_copy(out_v, o_hbm.at[:, pl.ds(base, _B_PER_W)])

    tail_tbl = weight[_NODES - 64:].reshape(8, 128)
    grouped = _relayout(w_t)
    out_t = _gather(grouped, tail_tbl, idx)
    return out_t.T


# speedup vs baseline: 1.1180x; 1.0005x over previous
import dataclasses
import functools

import jax
import jax.numpy as jnp
from jax import lax
from jax.experimental import pallas as pl
from jax.experimental.pallas import tpu as pltpu
from jax.experimental.pallas import tpu_sc as plsc

_BATCH = 16384
_GOAL_DIM = 16
_NODES = 1_000_000
_NUM_WORKERS = 32
_B_PER_W = _BATCH // _NUM_WORKERS
_CHUNK = 128
_LANES = 16
_GROUPS = _NODES // 8
_SLABS = (_NODES + 127) // 128
_SLABS_PER_W = (_SLABS + _NUM_WORKERS - 1) // _NUM_WORKERS


def _mesh_and_params():
    mesh = plsc.VectorSubcoreMesh(core_axis_name="c", subcore_axis_name="s")
    cp = pltpu.CompilerParams()
    if "needs_layout_passes" in pltpu.CompilerParams.__dataclass_fields__:
        cp = dataclasses.replace(cp, needs_layout_passes=False)
    return mesh, cp


def kernel(node_ids, weight):
    idx = node_ids.astype(jnp.int32)
    w_t = weight.T
    mesh, cp = _mesh_and_params()

    @functools.partial(
        pl.kernel,
        mesh=mesh,
        compiler_params=cp,
        out_type=jax.ShapeDtypeStruct((_GROUPS, 128), jnp.float32),
        scratch_types=[
            pltpu.VMEM((_GOAL_DIM, 128), jnp.float32),
            pltpu.VMEM((_GOAL_DIM, 128), jnp.float32),
            pltpu.VMEM((_GOAL_DIM, 128), jnp.float32),
            pltpu.VMEM((_GOAL_DIM, 128), jnp.float32),
            pltpu.SemaphoreType.DMA,
            pltpu.SemaphoreType.DMA,
            pltpu.SemaphoreType.DMA,
            pltpu.SemaphoreType.DMA,
        ],
    )
    def _relayout(w_hbm, g_hbm, in0, in1, out0, out1, sem_i0, sem_i1,
                  sem_o0, sem_o1):
        wid = lax.axis_index("c") * 16 + lax.axis_index("s")
        k0 = wid * _SLABS_PER_W
        n_full = jnp.minimum(
            jnp.maximum(_SLABS - 1 - k0, 0), _SLABS_PER_W
        )
        lane = lax.iota(jnp.int32, _LANES)
        ins = (in0, in1)
        outs = (out0, out1)
        sem_ins = (sem_i0, sem_i1)
        sem_outs = (sem_o0, sem_o1)

        def start_in(k, p):
            pltpu.async_copy(
                w_hbm.at[:, pl.ds((k0 + k) * 128, 128)], ins[p], sem_ins[p]
            )

        def wait_in(k, p):
            pltpu.make_async_copy(
                w_hbm.at[:, pl.ds((k0 + k) * 128, 128)], ins[p], sem_ins[p]
            ).wait()

        def start_out(k, p):
            pltpu.async_copy(
                outs[p], g_hbm.at[pl.ds((k0 + k) * 16, 16), :], sem_outs[p]
            )

        def wait_out(k, p):
            pltpu.make_async_copy(
                outs[p], g_hbm.at[pl.ds((k0 + k) * 16, 16), :], sem_outs[p]
            ).wait()

        def transpose_slab(buf, out_b):
            for g in range(16):
                for s in range(8):
                    out_b[g, pl.ds(s * _LANES, _LANES)] = plsc.load_gather(
                        buf, [lane, jnp.full((_LANES,), 8 * g + s, jnp.int32)]
                    )

        def step(k, p):
            @pl.when(k < n_full)
            def _():
                wait_in(k, p)

                @pl.when(k >= 2)
                def _():
                    wait_out(k - 2, p)

                transpose_slab(ins[p], outs[p])
                start_out(k, p)

                @pl.when(k + 2 < n_full)
                def _():
                    start_in(k + 2, p)

        @pl.when(n_full > 0)
        def _():
            start_in(0, 0)

            @pl.when(n_full > 1)
            def _():
                start_in(1, 1)

            @pl.loop(0, _SLABS_PER_W, step=2)
            def _(k):
                step(k, 0)
                step(k + 1, 1)

            @pl.when(n_full >= 2)
            def _():
                wait_out(n_full - 2, 1)

            @pl.when(n_full >= 1)
            def _():
                wait_out(n_full - 1, 0)


    @functools.partial(
        pl.kernel,
        mesh=mesh,
        compiler_params=cp,
        out_type=jax.ShapeDtypeStruct((_GOAL_DIM, _BATCH), jnp.float32),
        scratch_types=[
            pltpu.VMEM((_B_PER_W,), jnp.int32),
            pltpu.VMEM((_B_PER_W,), jnp.int32),
            pltpu.VMEM((_B_PER_W, 128), jnp.float32),
            pltpu.VMEM((_GOAL_DIM, _B_PER_W), jnp.float32),
            pltpu.VMEM((8, 128), jnp.float32),
            pltpu.SemaphoreType.DMA,
        ],
    )
    def _gather(g_hbm, t_hbm, i_hbm, o_hbm, idx_v, grp_v, blk_v, out_v,
                tail_sp, sem):
        wid = lax.axis_index("c") * 16 + lax.axis_index("s")
        base = wid * _B_PER_W

        pltpu.sync_copy(t_hbm, tail_sp)
        pltpu.sync_copy(i_hbm.at[pl.ds(base, _B_PER_W)], idx_v)

        @pl.loop(0, _B_PER_W, step=_LANES)
        def _(c):
            grp_v[pl.ds(c, _LANES)] = idx_v[pl.ds(c, _LANES)] >> 3

        @pl.loop(0, _B_PER_W, step=_CHUNK)
        def _(c):
            pltpu.async_copy(
                g_hbm.at[grp_v.at[pl.ds(c, _CHUNK)]],
                blk_v.at[pl.ds(c, _CHUNK), :],
                sem,
            )

        @pl.loop(0, _B_PER_W, step=_CHUNK)
        def _(c):
            pltpu.make_async_copy(
                g_hbm.at[grp_v.at[pl.ds(c, _CHUNK)]],
                blk_v.at[pl.ds(c, _CHUNK), :],
                sem,
            ).wait()

        lane = lax.iota(jnp.int32, _LANES)

        @pl.loop(0, _B_PER_W, step=_LANES)
        def _(c):
            b_vec = c + lane
            grp_c = grp_v[pl.ds(c, _LANES)]
            is_tail = grp_c >= _GROUPS - 8
            tg = jnp.where(is_tail, grp_c - (_GROUPS - 8), 0)
            sub16 = (idx_v[pl.ds(c, _LANES)] & 7) * 16
            for l in range(_LANES):
                main = plsc.load_gather(blk_v, [b_vec, sub16 + l])
                tailv = plsc.load_gather(tail_sp, [tg, sub16 + l])
                out_v[l, pl.ds(c, _LANES)] = jnp.where(is_tail, tailv, main)

        pltpu.sync_copy(out_v, o_hbm.at[:, pl.ds(base, _B_PER_W)])

    tail_tbl = weight[_NODES - 64:].reshape(8, 128)
    grouped = _relayout(w_t)
    out_t = _gather(grouped, tail_tbl, idx)
    return out_t.T
